# baseline (speedup 1.0000x reference)
import functools

import jax
import jax.numpy as jnp
from jax.experimental import pallas as pl
from jax.experimental.pallas import tpu as pltpu
from jax.experimental.pallas import tpu_sc as plsc

N_EXP = 8
TOP_K = 2
D_MODEL = 1024
D_FF = 2048
T = 2048
CAP = 640
NROW = 9 * CAP
TRASH = NROW - 1
NEG = -1e30


def _router_score_body(x_ref, gw_ref, e1_ref, e2_ref, g0_ref, g1_ref):
    logits = jax.lax.dot_general(
        x_ref[...].astype(jnp.bfloat16), gw_ref[...].astype(jnp.bfloat16),
        (((1,), (0,)), ((), ())),
        preferred_element_type=jnp.float32)
    lane = jax.lax.broadcasted_iota(jnp.int32, (T, 128), 1)
    lg = jnp.where(lane < N_EXP, logits, NEG)
    v1 = jnp.max(lg, axis=1, keepdims=True)
    e1 = jnp.min(jnp.where(lg == v1, lane, 127), axis=1, keepdims=True)
    lg2 = jnp.where(lane == e1, NEG, lg)
    v2 = jnp.max(lg2, axis=1, keepdims=True)
    e2 = jnp.min(jnp.where(lg2 == v2, lane, 127), axis=1, keepdims=True)
    g0 = jax.nn.sigmoid(v1 - v2)
    e1_ref[...] = e1
    e2_ref[...] = e2
    g0_ref[...] = g0
    g1_ref[...] = 1.0 - g0


_router_score = pl.pallas_call(
    _router_score_body,
    out_shape=(
        jax.ShapeDtypeStruct((T, 1), jnp.int32),
        jax.ShapeDtypeStruct((T, 1), jnp.int32),
        jax.ShapeDtypeStruct((T, 1), jnp.float32),
        jax.ShapeDtypeStruct((T, 1), jnp.float32),
    ),
)


def _router_prefix_body(e1_ref, e2_ref, g0_ref, g1_ref,
                        ds0_ref, ds1_ref, cs0_ref, cs1_ref,
                        gg0_ref, gg1_ref):
    E1 = e1_ref[...]
    E2 = e2_ref[...]
    r1 = jax.lax.broadcasted_iota(jnp.int32, (128, 128), 0)
    c1 = jax.lax.broadcasted_iota(jnp.int32, (128, 128), 1)
    U = (r1 < c1).astype(jnp.float32)
    r2 = jax.lax.broadcasted_iota(jnp.int32, (16, 16), 0)
    c2 = jax.lax.broadcasted_iota(jnp.int32, (16, 16), 1)
    TL = (c2 < r2).astype(jnp.float32)

    Ws = []
    Scols = []
    for e in range(N_EXP):
        A = (E1 == e).astype(jnp.float32) + (E2 == e).astype(jnp.float32)
        Ws.append(jax.lax.dot_general(
            A, U, (((1,), (0,)), ((), ())),
            preferred_element_type=jnp.float32))
        Scols.append(jnp.sum(A, axis=1, keepdims=True))
    S = jnp.concatenate(Scols, axis=1)
    CP = jax.lax.dot_general(
        TL, S, (((1,), (0,)), ((), ())),
        preferred_element_type=jnp.float32)

    pos0 = jnp.zeros((16, 128), jnp.float32)
    pos1 = jnp.zeros((16, 128), jnp.float32)
    for e in range(N_EXP):
        cnt = Ws[e] + CP[:, e:e + 1]
        pos0 = pos0 + jnp.where(E1 == e, cnt, 0.0)
        pos1 = pos1 + jnp.where(E2 == e, cnt, 0.0)
    p0 = pos0.astype(jnp.int32)
    p1 = pos1.astype(jnp.int32)
    v0 = p0 < CAP
    v1 = p1 < CAP
    s0 = E1 * CAP + jnp.minimum(p0, CAP - 1)
    s1 = E2 * CAP + jnp.minimum(p1, CAP - 1)
    ds0_ref[...] = jnp.where(v0, s0, TRASH)
    ds1_ref[...] = jnp.where(v1, s1, TRASH)
    cs0_ref[...] = s0
    cs1_ref[...] = s1
    gg0_ref[...] = jnp.where(v0, g0_ref[...], 0.0)
    gg1_ref[...] = jnp.where(v1, g1_ref[...], 0.0)


_router_prefix = pl.pallas_call(
    _router_prefix_body,
    out_shape=(
        jax.ShapeDtypeStruct((16, 128), jnp.int32),
        jax.ShapeDtypeStruct((16, 128), jnp.int32),
        jax.ShapeDtypeStruct((16, 128), jnp.int32),
        jax.ShapeDtypeStruct((16, 128), jnp.int32),
        jax.ShapeDtypeStruct((16, 128), jnp.float32),
        jax.ShapeDtypeStruct((16, 128), jnp.float32),
    ),
)


_NC = 2
_NW = 32
_TPW = T // _NW


@functools.lru_cache(maxsize=None)
def _sc_dispatch():
    mesh = plsc.VectorSubcoreMesh(core_axis_name="c", subcore_axis_name="s")

    @functools.partial(
        pl.kernel,
        out_type=jax.ShapeDtypeStruct((NROW, D_MODEL), jnp.float32),
        mesh=mesh,
        scratch_types=[
            pltpu.VMEM((_TPW,), jnp.int32),
            pltpu.VMEM((_TPW,), jnp.int32),
            pltpu.VMEM((_TPW, D_MODEL), jnp.float32),
            pltpu.SemaphoreType.DMA,
        ],
    )
    def _dispatch(x_hbm, ds0_hbm, ds1_hbm, out_hbm, idx0_v, idx1_v, rows_v, sem):
        wid = jax.lax.axis_index("s") * _NC + jax.lax.axis_index("c")
        base = wid * _TPW
        pltpu.sync_copy(x_hbm.at[pl.ds(base, _TPW)], rows_v)
        pltpu.sync_copy(ds0_hbm.at[pl.ds(base, _TPW)], idx0_v)
        pltpu.sync_copy(ds1_hbm.at[pl.ds(base, _TPW)], idx1_v)
        pltpu.async_copy(rows_v, out_hbm.at[idx0_v], sem).wait()
        pltpu.async_copy(rows_v, out_hbm.at[idx1_v], sem).wait()

    return _dispatch


_BF = 1024
_NF = D_FF // _BF


def _glu_body(x_ref, wa_ref, wb_ref, wo_ref, o_ref):
    f = pl.program_id(1)
    xb = x_ref[...].astype(jnp.bfloat16)
    wa = wa_ref[0, 0].astype(jnp.bfloat16)
    wb = wb_ref[0, 0].astype(jnp.bfloat16)
    hA = jax.lax.dot_general(xb, wa, (((1,), (1,)), ((), ())),
                             preferred_element_type=jnp.float32)
    hB = jax.lax.dot_general(xb, wb, (((1,), (1,)), ((), ())),
                             preferred_element_type=jnp.float32)
    act = 0.5 * hA * (1.0 + jax.lax.erf(hA * 0.7071067811865476)) * hB
    wo = wo_ref[0].astype(jnp.bfloat16)
    p = jax.lax.dot_general(act.astype(jnp.bfloat16), wo,
                            (((1,), (1,)), ((), ())),
                            preferred_element_type=jnp.float32)

    @pl.when(f == 0)
    def _():
        o_ref[...] = p

    @pl.when(f != 0)
    def _():
        o_ref[...] += p


_glu = pl.pallas_call(
    _glu_body,
    grid=(N_EXP, _NF),
    in_specs=[
        pl.BlockSpec((CAP, D_MODEL), lambda e, f: (e, 0)),
        pl.BlockSpec((1, 1, _BF, D_MODEL), lambda e, f: (e, 0, f, 0)),
        pl.BlockSpec((1, 1, _BF, D_MODEL), lambda e, f: (e, 1, f, 0)),
        pl.BlockSpec((1, D_MODEL, _BF), lambda e, f: (e, 0, f)),
    ],
    out_specs=pl.BlockSpec((CAP, D_MODEL), lambda e, f: (e, 0)),
    out_shape=jax.ShapeDtypeStruct((N_EXP * CAP, D_MODEL), jnp.float32),
    compiler_params=pltpu.CompilerParams(
        dimension_semantics=("arbitrary", "arbitrary"),
        vmem_limit_bytes=120 * 1024 * 1024),
)


_CH = 32


@functools.lru_cache(maxsize=None)
def _sc_combine():
    mesh = plsc.VectorSubcoreMesh(core_axis_name="c", subcore_axis_name="s")

    @functools.partial(
        pl.kernel,
        out_type=jax.ShapeDtypeStruct((T, D_MODEL), jnp.float32),
        mesh=mesh,
        scratch_types=[
            pltpu.VMEM((_CH,), jnp.int32),
            pltpu.VMEM((_CH,), jnp.int32),
            pltpu.VMEM((_CH, 16), jnp.float32),
            pltpu.VMEM((_CH, 16), jnp.float32),
            pltpu.VMEM((_CH, D_MODEL), jnp.float32),
            pltpu.VMEM((_CH, D_MODEL), jnp.float32),
            pltpu.SemaphoreType.DMA,
        ],
    )
    def _combine(eo_hbm, cs0_hbm, cs1_hbm, g0_hbm, g1_hbm, out_hbm,
                 i0_v, i1_v, g0_v, g1_v, b0_v, b1_v, sem):
        wid = jax.lax.axis_index("s") * _NC + jax.lax.axis_index("c")
        for half in range(_TPW // _CH):
            base = wid * _TPW + half * _CH
            pltpu.sync_copy(cs0_hbm.at[pl.ds(base, _CH)], i0_v)
            pltpu.sync_copy(cs1_hbm.at[pl.ds(base, _CH)], i1_v)
            pltpu.sync_copy(g0_hbm.at[pl.ds(base, _CH)], g0_v)
            pltpu.sync_copy(g1_hbm.at[pl.ds(base, _CH)], g1_v)
            pltpu.async_copy(eo_hbm.at[i0_v], b0_v, sem).wait()
            pltpu.async_copy(eo_hbm.at[i1_v], b1_v, sem).wait()

            def row_body(r, carry):
                ga = g0_v[r, :]
                gb = g1_v[r, :]
                for v in range(D_MODEL // 16):
                    sl = pl.ds(v * 16, 16)
                    b0_v[r, sl] = ga * b0_v[r, sl] + gb * b1_v[r, sl]
                return carry

            jax.lax.fori_loop(0, _CH, row_body, 0)
            pltpu.sync_copy(b0_v, out_hbm.at[pl.ds(base, _CH)])

    return _combine


def kernel(hidden_states, gate_w, w_in, w_out):
    x2d = hidden_states.reshape(T, D_MODEL)
    gwt = jnp.zeros((D_MODEL, 128), jnp.float32).at[:, :N_EXP].set(gate_w.T)
    e1, e2, g0, g1 = _router_score(x2d, gwt)
    ds0, ds1, cs0, cs1, gg0, gg1 = _router_prefix(
        e1.reshape(16, 128), e2.reshape(16, 128),
        g0.reshape(16, 128), g1.reshape(16, 128))
    expx = _sc_dispatch()(x2d, ds0.reshape(T), ds1.reshape(T))
    w_in4 = w_in.reshape(N_EXP, 2, D_FF, D_MODEL)
    eo = _glu(expx, w_in4, w_in4, w_out)
    g0x = jnp.broadcast_to(gg0.reshape(T, 1), (T, 16))
    g1x = jnp.broadcast_to(gg1.reshape(T, 1), (T, 16))
    out = _sc_combine()(eo, cs0.reshape(T), cs1.reshape(T), g0x, g1x)
    return out.reshape(1, T, D_MODEL)


# speedup vs baseline: 1.0428x; 1.0428x over previous
import functools

import jax
import jax.numpy as jnp
from jax.experimental import pallas as pl
from jax.experimental.pallas import tpu as pltpu
from jax.experimental.pallas import tpu_sc as plsc

N_EXP = 8
TOP_K = 2
D_MODEL = 1024
D_FF = 2048
T = 2048
CAP = 640
NROW = 9 * CAP
TRASH = NROW - 1
NEG = -1e30


def _router_score_body(x_ref, gw_ref, e1_ref, e2_ref, g0_ref, g1_ref):
    logits = jax.lax.dot_general(
        x_ref[...].astype(jnp.bfloat16), gw_ref[...].astype(jnp.bfloat16),
        (((1,), (0,)), ((), ())),
        preferred_element_type=jnp.float32)
    lane = jax.lax.broadcasted_iota(jnp.int32, (T, 128), 1)
    lg = jnp.where(lane < N_EXP, logits, NEG)
    v1 = jnp.max(lg, axis=1, keepdims=True)
    e1 = jnp.min(jnp.where(lg == v1, lane, 127), axis=1, keepdims=True)
    lg2 = jnp.where(lane == e1, NEG, lg)
    v2 = jnp.max(lg2, axis=1, keepdims=True)
    e2 = jnp.min(jnp.where(lg2 == v2, lane, 127), axis=1, keepdims=True)
    g0 = jax.nn.sigmoid(v1 - v2)
    e1_ref[...] = e1
    e2_ref[...] = e2
    g0_ref[...] = g0
    g1_ref[...] = 1.0 - g0


_router_score = pl.pallas_call(
    _router_score_body,
    out_shape=(
        jax.ShapeDtypeStruct((T, 1), jnp.int32),
        jax.ShapeDtypeStruct((T, 1), jnp.int32),
        jax.ShapeDtypeStruct((T, 1), jnp.float32),
        jax.ShapeDtypeStruct((T, 1), jnp.float32),
    ),
)


def _router_prefix_body(e1_ref, e2_ref, g0_ref, g1_ref,
                        ds0_ref, ds1_ref, cs0_ref, cs1_ref,
                        gg0_ref, gg1_ref):
    E1 = e1_ref[...]
    E2 = e2_ref[...]
    r1 = jax.lax.broadcasted_iota(jnp.int32, (128, 128), 0)
    c1 = jax.lax.broadcasted_iota(jnp.int32, (128, 128), 1)
    U = (r1 < c1).astype(jnp.float32)
    r2 = jax.lax.broadcasted_iota(jnp.int32, (16, 16), 0)
    c2 = jax.lax.broadcasted_iota(jnp.int32, (16, 16), 1)
    TL = (c2 < r2).astype(jnp.float32)

    Ws = []
    Scols = []
    for e in range(N_EXP):
        A = (E1 == e).astype(jnp.float32) + (E2 == e).astype(jnp.float32)
        Ws.append(jax.lax.dot_general(
            A, U, (((1,), (0,)), ((), ())),
            preferred_element_type=jnp.float32))
        Scols.append(jnp.sum(A, axis=1, keepdims=True))
    S = jnp.concatenate(Scols, axis=1)
    CP = jax.lax.dot_general(
        TL, S, (((1,), (0,)), ((), ())),
        preferred_element_type=jnp.float32)

    pos0 = jnp.zeros((16, 128), jnp.float32)
    pos1 = jnp.zeros((16, 128), jnp.float32)
    for e in range(N_EXP):
        cnt = Ws[e] + CP[:, e:e + 1]
        pos0 = pos0 + jnp.where(E1 == e, cnt, 0.0)
        pos1 = pos1 + jnp.where(E2 == e, cnt, 0.0)
    p0 = pos0.astype(jnp.int32)
    p1 = pos1.astype(jnp.int32)
    v0 = p0 < CAP
    v1 = p1 < CAP
    s0 = E1 * CAP + jnp.minimum(p0, CAP - 1)
    s1 = E2 * CAP + jnp.minimum(p1, CAP - 1)
    ds0_ref[...] = jnp.where(v0, s0, TRASH)
    ds1_ref[...] = jnp.where(v1, s1, TRASH)
    cs0_ref[...] = s0
    cs1_ref[...] = s1
    gg0_ref[...] = jnp.where(v0, g0_ref[...], 0.0)
    gg1_ref[...] = jnp.where(v1, g1_ref[...], 0.0)


_router_prefix = pl.pallas_call(
    _router_prefix_body,
    out_shape=(
        jax.ShapeDtypeStruct((16, 128), jnp.int32),
        jax.ShapeDtypeStruct((16, 128), jnp.int32),
        jax.ShapeDtypeStruct((16, 128), jnp.int32),
        jax.ShapeDtypeStruct((16, 128), jnp.int32),
        jax.ShapeDtypeStruct((16, 128), jnp.float32),
        jax.ShapeDtypeStruct((16, 128), jnp.float32),
    ),
)


_NC = 2
_NW = 32
_TPW = T // _NW


@functools.lru_cache(maxsize=None)
def _sc_dispatch():
    mesh = plsc.VectorSubcoreMesh(core_axis_name="c", subcore_axis_name="s")

    @functools.partial(
        pl.kernel,
        out_type=jax.ShapeDtypeStruct((NROW, D_MODEL), jnp.float32),
        mesh=mesh,
        scratch_types=[
            pltpu.VMEM((_TPW,), jnp.int32),
            pltpu.VMEM((_TPW,), jnp.int32),
            pltpu.VMEM((_TPW, D_MODEL), jnp.float32),
            pltpu.SemaphoreType.DMA,
        ],
    )
    def _dispatch(x_hbm, ds0_hbm, ds1_hbm, out_hbm, idx0_v, idx1_v, rows_v, sem):
        wid = jax.lax.axis_index("s") * _NC + jax.lax.axis_index("c")
        base = wid * _TPW
        pltpu.sync_copy(x_hbm.at[pl.ds(base, _TPW)], rows_v)
        pltpu.sync_copy(ds0_hbm.at[pl.ds(base, _TPW)], idx0_v)
        pltpu.sync_copy(ds1_hbm.at[pl.ds(base, _TPW)], idx1_v)
        c0 = pltpu.async_copy(rows_v, out_hbm.at[idx0_v], sem)
        c1 = pltpu.async_copy(rows_v, out_hbm.at[idx1_v], sem)
        c0.wait()
        c1.wait()

    return _dispatch


_BF = 1024
_NF = D_FF // _BF


def _glu_body(x_ref, wa_ref, wb_ref, wo_ref, o_ref):
    f = pl.program_id(1)
    xb = x_ref[...].astype(jnp.bfloat16)
    wa = wa_ref[0, 0].astype(jnp.bfloat16)
    wb = wb_ref[0, 0].astype(jnp.bfloat16)
    hA = jax.lax.dot_general(xb, wa, (((1,), (1,)), ((), ())),
                             preferred_element_type=jnp.float32)
    hB = jax.lax.dot_general(xb, wb, (((1,), (1,)), ((), ())),
                             preferred_element_type=jnp.float32)
    act = 0.5 * hA * (1.0 + jax.lax.erf(hA * 0.7071067811865476)) * hB
    wo = wo_ref[0].astype(jnp.bfloat16)
    p = jax.lax.dot_general(act.astype(jnp.bfloat16), wo,
                            (((1,), (1,)), ((), ())),
                            preferred_element_type=jnp.float32)

    @pl.when(f == 0)
    def _():
        o_ref[...] = p

    @pl.when(f != 0)
    def _():
        o_ref[...] += p


_glu = pl.pallas_call(
    _glu_body,
    grid=(N_EXP, _NF),
    in_specs=[
        pl.BlockSpec((CAP, D_MODEL), lambda e, f: (e, 0)),
        pl.BlockSpec((1, 1, _BF, D_MODEL), lambda e, f: (e, 0, f, 0)),
        pl.BlockSpec((1, 1, _BF, D_MODEL), lambda e, f: (e, 1, f, 0)),
        pl.BlockSpec((1, D_MODEL, _BF), lambda e, f: (e, 0, f)),
    ],
    out_specs=pl.BlockSpec((CAP, D_MODEL), lambda e, f: (e, 0)),
    out_shape=jax.ShapeDtypeStruct((N_EXP * CAP, D_MODEL), jnp.float32),
    compiler_params=pltpu.CompilerParams(
        dimension_semantics=("arbitrary", "arbitrary"),
        vmem_limit_bytes=120 * 1024 * 1024),
)


_CH = 16


@functools.lru_cache(maxsize=None)
def _sc_combine():
    mesh = plsc.VectorSubcoreMesh(core_axis_name="c", subcore_axis_name="s")

    @functools.partial(
        pl.kernel,
        out_type=jax.ShapeDtypeStruct((T, D_MODEL), jnp.float32),
        mesh=mesh,
        scratch_types=[
            pltpu.VMEM((_TPW,), jnp.int32),
            pltpu.VMEM((_TPW,), jnp.int32),
            pltpu.VMEM((_TPW, 32), jnp.float32),
            pltpu.VMEM((2, _CH, D_MODEL), jnp.float32),
            pltpu.VMEM((2, _CH, D_MODEL), jnp.float32),
            [pltpu.SemaphoreType.DMA, pltpu.SemaphoreType.DMA],
        ],
    )
    def _combine(eo_hbm, cs0_hbm, cs1_hbm, g_hbm, out_hbm,
                 i0_v, i1_v, g_v, b0_v, b1_v, sems):
        wid = jax.lax.axis_index("s") * _NC + jax.lax.axis_index("c")
        base = wid * _TPW
        pltpu.sync_copy(cs0_hbm.at[pl.ds(base, _TPW)], i0_v)
        pltpu.sync_copy(cs1_hbm.at[pl.ds(base, _TPW)], i1_v)
        pltpu.sync_copy(g_hbm.at[pl.ds(base, _TPW)], g_v)
        nch = _TPW // _CH
        copies = []
        for h in range(nch):
            buf = h % 2
            c0 = pltpu.async_copy(
                eo_hbm.at[i0_v.at[pl.ds(h * _CH, _CH)]], b0_v.at[buf], sems[buf])
            c1 = pltpu.async_copy(
                eo_hbm.at[i1_v.at[pl.ds(h * _CH, _CH)]], b1_v.at[buf], sems[buf])
            copies.append((c0, c1))
            if h == 0:
                continue
            prev = h - 1
            pbuf = prev % 2
            copies[prev][0].wait()
            copies[prev][1].wait()

            def row_body(r, carry, pbuf=pbuf, prev=prev):
                ga = g_v[prev * _CH + r, pl.ds(0, 16)]
                gb = g_v[prev * _CH + r, pl.ds(16, 16)]
                for v in range(D_MODEL // 16):
                    sl = pl.ds(v * 16, 16)
                    b0_v[pbuf, r, sl] = (ga * b0_v[pbuf, r, sl]
                                         + gb * b1_v[pbuf, r, sl])
                return carry

            jax.lax.fori_loop(0, _CH, row_body, 0)
            pltpu.sync_copy(b0_v.at[pbuf],
                            out_hbm.at[pl.ds(base + prev * _CH, _CH)])
        last = nch - 1
        lbuf = last % 2
        copies[last][0].wait()
        copies[last][1].wait()

        def row_body_l(r, carry):
            ga = g_v[last * _CH + r, pl.ds(0, 16)]
            gb = g_v[last * _CH + r, pl.ds(16, 16)]
            for v in range(D_MODEL // 16):
                sl = pl.ds(v * 16, 16)
                b0_v[lbuf, r, sl] = (ga * b0_v[lbuf, r, sl]
                                     + gb * b1_v[lbuf, r, sl])
            return carry

        jax.lax.fori_loop(0, _CH, row_body_l, 0)
        pltpu.sync_copy(b0_v.at[lbuf],
                        out_hbm.at[pl.ds(base + last * _CH, _CH)])

    return _combine


def kernel(hidden_states, gate_w, w_in, w_out):
    x2d = hidden_states.reshape(T, D_MODEL)
    gwt = jnp.zeros((D_MODEL, 128), jnp.float32).at[:, :N_EXP].set(gate_w.T)
    e1, e2, g0, g1 = _router_score(x2d, gwt)
    ds0, ds1, cs0, cs1, gg0, gg1 = _router_prefix(
        e1.reshape(16, 128), e2.reshape(16, 128),
        g0.reshape(16, 128), g1.reshape(16, 128))
    expx = _sc_dispatch()(x2d, ds0.reshape(T), ds1.reshape(T))
    w_in4 = w_in.reshape(N_EXP, 2, D_FF, D_MODEL)
    eo = _glu(expx, w_in4, w_in4, w_out)
    gx = jnp.concatenate(
        [jnp.broadcast_to(gg0.reshape(T, 1), (T, 16)),
         jnp.broadcast_to(gg1.reshape(T, 1), (T, 16))], axis=1)
    out = _sc_combine()(eo, cs0.reshape(T), cs1.reshape(T), gx)
    return out.reshape(1, T, D_MODEL)


# speedup vs baseline: 1.0600x; 1.0165x over previous
import functools

import jax
import jax.numpy as jnp
from jax.experimental import pallas as pl
from jax.experimental.pallas import tpu as pltpu
from jax.experimental.pallas import tpu_sc as plsc

N_EXP = 8
TOP_K = 2
D_MODEL = 1024
D_FF = 2048
T = 2048
CAP = 640
NROW = 9 * CAP
TRASH = NROW - 1
NEG = -1e30


def _router_body(x_ref, gw_ref,
                 ds0_ref, ds1_ref, cs0_ref, cs1_ref,
                 gg0_ref, gg1_ref):
    logits = jax.lax.dot_general(
        x_ref[...].astype(jnp.bfloat16), gw_ref[...].astype(jnp.bfloat16),
        (((1,), (0,)), ((), ())),
        preferred_element_type=jnp.float32)
    lane = jax.lax.broadcasted_iota(jnp.int32, (T, 128), 1)
    lg = jnp.where(lane < N_EXP, logits, NEG)
    v1 = jnp.max(lg, axis=1, keepdims=True)
    e1 = jnp.min(jnp.where(lg == v1, lane, 127), axis=1, keepdims=True)
    lg2 = jnp.where(lane == e1, NEG, lg)
    v2 = jnp.max(lg2, axis=1, keepdims=True)
    e2 = jnp.min(jnp.where(lg2 == v2, lane, 127), axis=1, keepdims=True)
    g0 = jax.nn.sigmoid(v1 - v2)
    E1 = e1.reshape(16, 128)
    E2 = e2.reshape(16, 128)
    G0 = g0.reshape(16, 128)
    G1 = (1.0 - g0).reshape(16, 128)
    r1 = jax.lax.broadcasted_iota(jnp.int32, (128, 128), 0)
    c1 = jax.lax.broadcasted_iota(jnp.int32, (128, 128), 1)
    U = (r1 < c1).astype(jnp.float32)
    r2 = jax.lax.broadcasted_iota(jnp.int32, (16, 16), 0)
    c2 = jax.lax.broadcasted_iota(jnp.int32, (16, 16), 1)
    TL = (c2 < r2).astype(jnp.float32)

    Ws = []
    Scols = []
    for e in range(N_EXP):
        A = (E1 == e).astype(jnp.float32) + (E2 == e).astype(jnp.float32)
        Ws.append(jax.lax.dot_general(
            A, U, (((1,), (0,)), ((), ())),
            preferred_element_type=jnp.float32))
        Scols.append(jnp.sum(A, axis=1, keepdims=True))
    S = jnp.concatenate(Scols, axis=1)
    CP = jax.lax.dot_general(
        TL, S, (((1,), (0,)), ((), ())),
        preferred_element_type=jnp.float32)

    pos0 = jnp.zeros((16, 128), jnp.float32)
    pos1 = jnp.zeros((16, 128), jnp.float32)
    for e in range(N_EXP):
        cnt = Ws[e] + CP[:, e:e + 1]
        pos0 = pos0 + jnp.where(E1 == e, cnt, 0.0)
        pos1 = pos1 + jnp.where(E2 == e, cnt, 0.0)
    p0 = pos0.astype(jnp.int32)
    p1 = pos1.astype(jnp.int32)
    v0 = p0 < CAP
    v1 = p1 < CAP
    s0 = E1 * CAP + jnp.minimum(p0, CAP - 1)
    s1 = E2 * CAP + jnp.minimum(p1, CAP - 1)
    ds0_ref[...] = jnp.where(v0, s0, TRASH)
    ds1_ref[...] = jnp.where(v1, s1, TRASH)
    cs0_ref[...] = s0
    cs1_ref[...] = s1
    gg0_ref[...] = jnp.where(v0, G0, 0.0)
    gg1_ref[...] = jnp.where(v1, G1, 0.0)


_router = pl.pallas_call(
    _router_body,
    out_shape=(
        jax.ShapeDtypeStruct((16, 128), jnp.int32),
        jax.ShapeDtypeStruct((16, 128), jnp.int32),
        jax.ShapeDtypeStruct((16, 128), jnp.int32),
        jax.ShapeDtypeStruct((16, 128), jnp.int32),
        jax.ShapeDtypeStruct((16, 128), jnp.float32),
        jax.ShapeDtypeStruct((16, 128), jnp.float32),
    ),
)


_NC = 2
_NW = 32
_TPW = T // _NW


@functools.lru_cache(maxsize=None)
def _sc_dispatch():
    mesh = plsc.VectorSubcoreMesh(core_axis_name="c", subcore_axis_name="s")

    @functools.partial(
        pl.kernel,
        out_type=jax.ShapeDtypeStruct((NROW, D_MODEL), jnp.float32),
        mesh=mesh,
        scratch_types=[
            pltpu.VMEM((_TPW,), jnp.int32),
            pltpu.VMEM((_TPW,), jnp.int32),
            pltpu.VMEM((_TPW, D_MODEL), jnp.float32),
            pltpu.SemaphoreType.DMA,
        ],
    )
    def _dispatch(x_hbm, ds0_hbm, ds1_hbm, out_hbm, idx0_v, idx1_v, rows_v, sem):
        wid = jax.lax.axis_index("s") * _NC + jax.lax.axis_index("c")
        base = wid * _TPW
        pltpu.sync_copy(x_hbm.at[pl.ds(base, _TPW)], rows_v)
        pltpu.sync_copy(ds0_hbm.at[pl.ds(base, _TPW)], idx0_v)
        pltpu.sync_copy(ds1_hbm.at[pl.ds(base, _TPW)], idx1_v)
        c0 = pltpu.async_copy(rows_v, out_hbm.at[idx0_v], sem)
        c1 = pltpu.async_copy(rows_v, out_hbm.at[idx1_v], sem)
        c0.wait()
        c1.wait()

    return _dispatch


_BF = 1024
_NF = D_FF // _BF


def _glu_body(x_ref, wa_ref, wb_ref, wo_ref, o_ref):
    f = pl.program_id(1)
    xb = x_ref[...].astype(jnp.bfloat16)
    wa = wa_ref[0, 0].astype(jnp.bfloat16)
    wb = wb_ref[0, 0].astype(jnp.bfloat16)
    hA = jax.lax.dot_general(xb, wa, (((1,), (1,)), ((), ())),
                             preferred_element_type=jnp.float32)
    hB = jax.lax.dot_general(xb, wb, (((1,), (1,)), ((), ())),
                             preferred_element_type=jnp.float32)
    act = 0.5 * hA * (1.0 + jax.lax.erf(hA * 0.7071067811865476)) * hB
    wo = wo_ref[0].astype(jnp.bfloat16)
    p = jax.lax.dot_general(act.astype(jnp.bfloat16), wo,
                            (((1,), (1,)), ((), ())),
                            preferred_element_type=jnp.float32)

    @pl.when(f == 0)
    def _():
        o_ref[...] = p

    @pl.when(f != 0)
    def _():
        o_ref[...] += p


_glu = pl.pallas_call(
    _glu_body,
    grid=(N_EXP, _NF),
    in_specs=[
        pl.BlockSpec((CAP, D_MODEL), lambda e, f: (e, 0)),
        pl.BlockSpec((1, 1, _BF, D_MODEL), lambda e, f: (e, 0, f, 0)),
        pl.BlockSpec((1, 1, _BF, D_MODEL), lambda e, f: (e, 1, f, 0)),
        pl.BlockSpec((1, D_MODEL, _BF), lambda e, f: (e, 0, f)),
    ],
    out_specs=pl.BlockSpec((CAP, D_MODEL), lambda e, f: (e, 0)),
    out_shape=jax.ShapeDtypeStruct((N_EXP * CAP, D_MODEL), jnp.float32),
    compiler_params=pltpu.CompilerParams(
        dimension_semantics=("arbitrary", "arbitrary"),
        vmem_limit_bytes=120 * 1024 * 1024),
)


_CH = 16


@functools.lru_cache(maxsize=None)
def _sc_combine():
    mesh = plsc.VectorSubcoreMesh(core_axis_name="c", subcore_axis_name="s")

    @functools.partial(
        pl.kernel,
        out_type=jax.ShapeDtypeStruct((T, D_MODEL), jnp.float32),
        mesh=mesh,
        scratch_types=[
            pltpu.VMEM((_TPW,), jnp.int32),
            pltpu.VMEM((_TPW,), jnp.int32),
            pltpu.VMEM((_TPW, 32), jnp.float32),
            pltpu.VMEM((2, _CH, D_MODEL), jnp.float32),
            pltpu.VMEM((2, _CH, D_MODEL), jnp.float32),
            [pltpu.SemaphoreType.DMA, pltpu.SemaphoreType.DMA],
        ],
    )
    def _combine(eo_hbm, cs0_hbm, cs1_hbm, g_hbm, out_hbm,
                 i0_v, i1_v, g_v, b0_v, b1_v, sems):
        wid = jax.lax.axis_index("s") * _NC + jax.lax.axis_index("c")
        base = wid * _TPW
        pltpu.sync_copy(cs0_hbm.at[pl.ds(base, _TPW)], i0_v)
        pltpu.sync_copy(cs1_hbm.at[pl.ds(base, _TPW)], i1_v)
        pltpu.sync_copy(g_hbm.at[pl.ds(base, _TPW)], g_v)
        nch = _TPW // _CH
        copies = []
        for h in range(nch):
            buf = h % 2
            c0 = pltpu.async_copy(
                eo_hbm.at[i0_v.at[pl.ds(h * _CH, _CH)]], b0_v.at[buf], sems[buf])
            c1 = pltpu.async_copy(
                eo_hbm.at[i1_v.at[pl.ds(h * _CH, _CH)]], b1_v.at[buf], sems[buf])
            copies.append((c0, c1))
            if h == 0:
                continue
            prev = h - 1
            pbuf = prev % 2
            copies[prev][0].wait()
            copies[prev][1].wait()

            def row_body(r, carry, pbuf=pbuf, prev=prev):
                ga = g_v[prev * _CH + r, pl.ds(0, 16)]
                gb = g_v[prev * _CH + r, pl.ds(16, 16)]
                for v in range(D_MODEL // 16):
                    sl = pl.ds(v * 16, 16)
                    b0_v[pbuf, r, sl] = (ga * b0_v[pbuf, r, sl]
                                         + gb * b1_v[pbuf, r, sl])
                return carry

            jax.lax.fori_loop(0, _CH, row_body, 0)
            pltpu.sync_copy(b0_v.at[pbuf],
                            out_hbm.at[pl.ds(base + prev * _CH, _CH)])
        last = nch - 1
        lbuf = last % 2
        copies[last][0].wait()
        copies[last][1].wait()

        def row_body_l(r, carry):
            ga = g_v[last * _CH + r, pl.ds(0, 16)]
            gb = g_v[last * _CH + r, pl.ds(16, 16)]
            for v in range(D_MODEL // 16):
                sl = pl.ds(v * 16, 16)
                b0_v[lbuf, r, sl] = (ga * b0_v[lbuf, r, sl]
                                     + gb * b1_v[lbuf, r, sl])
            return carry

        jax.lax.fori_loop(0, _CH, row_body_l, 0)
        pltpu.sync_copy(b0_v.at[lbuf],
                        out_hbm.at[pl.ds(base + last * _CH, _CH)])

    return _combine


def kernel(hidden_states, gate_w, w_in, w_out):
    x2d = hidden_states.reshape(T, D_MODEL)
    gwt = jnp.zeros((D_MODEL, 128), jnp.float32).at[:, :N_EXP].set(gate_w.T)
    ds0, ds1, cs0, cs1, gg0, gg1 = _router(x2d, gwt)
    expx = _sc_dispatch()(x2d, ds0.reshape(T), ds1.reshape(T))
    w_in4 = w_in.reshape(N_EXP, 2, D_FF, D_MODEL)
    eo = _glu(expx, w_in4, w_in4, w_out)
    gx = jnp.concatenate(
        [jnp.broadcast_to(gg0.reshape(T, 1), (T, 16)),
         jnp.broadcast_to(gg1.reshape(T, 1), (T, 16))], axis=1)
    out = _sc_combine()(eo, cs0.reshape(T), cs1.reshape(T), gx)
    return out.reshape(1, T, D_MODEL)


# speedup vs baseline: 1.0678x; 1.0073x over previous
import functools

import jax
import jax.numpy as jnp
from jax.experimental import pallas as pl
from jax.experimental.pallas import tpu as pltpu
from jax.experimental.pallas import tpu_sc as plsc

N_EXP = 8
TOP_K = 2
D_MODEL = 1024
D_FF = 2048
T = 2048
CAP = 640
NROW = 9 * CAP
TRASH = NROW - 1
NEG = -1e30


def _router_body(x_ref, gw_ref,
                 ds0_ref, ds1_ref, cs0_ref, cs1_ref,
                 gg0_ref, gg1_ref):
    logits = jax.lax.dot_general(
        x_ref[...].astype(jnp.bfloat16), gw_ref[...].astype(jnp.bfloat16),
        (((1,), (0,)), ((), ())),
        preferred_element_type=jnp.float32)
    lane = jax.lax.broadcasted_iota(jnp.int32, (T, 128), 1)
    lg = jnp.where(lane < N_EXP, logits, NEG)
    v1 = jnp.max(lg, axis=1, keepdims=True)
    e1 = jnp.min(jnp.where(lg == v1, lane, 127), axis=1, keepdims=True)
    lg2 = jnp.where(lane == e1, NEG, lg)
    v2 = jnp.max(lg2, axis=1, keepdims=True)
    e2 = jnp.min(jnp.where(lg2 == v2, lane, 127), axis=1, keepdims=True)
    g0 = jax.nn.sigmoid(v1 - v2)
    E1 = e1.reshape(16, 128)
    E2 = e2.reshape(16, 128)
    G0 = g0.reshape(16, 128)
    G1 = (1.0 - g0).reshape(16, 128)
    r1 = jax.lax.broadcasted_iota(jnp.int32, (128, 128), 0)
    c1 = jax.lax.broadcasted_iota(jnp.int32, (128, 128), 1)
    U = (r1 < c1).astype(jnp.float32)
    r2 = jax.lax.broadcasted_iota(jnp.int32, (16, 16), 0)
    c2 = jax.lax.broadcasted_iota(jnp.int32, (16, 16), 1)
    TL = (c2 < r2).astype(jnp.float32)

    Ws = []
    Scols = []
    for e in range(N_EXP):
        A = (E1 == e).astype(jnp.float32) + (E2 == e).astype(jnp.float32)
        Ws.append(jax.lax.dot_general(
            A, U, (((1,), (0,)), ((), ())),
            preferred_element_type=jnp.float32))
        Scols.append(jnp.sum(A, axis=1, keepdims=True))
    S = jnp.concatenate(Scols, axis=1)
    CP = jax.lax.dot_general(
        TL, S, (((1,), (0,)), ((), ())),
        preferred_element_type=jnp.float32)

    pos0 = jnp.zeros((16, 128), jnp.float32)
    pos1 = jnp.zeros((16, 128), jnp.float32)
    for e in range(N_EXP):
        cnt = Ws[e] + CP[:, e:e + 1]
        pos0 = pos0 + jnp.where(E1 == e, cnt, 0.0)
        pos1 = pos1 + jnp.where(E2 == e, cnt, 0.0)
    p0 = pos0.astype(jnp.int32)
    p1 = pos1.astype(jnp.int32)
    v0 = p0 < CAP
    v1 = p1 < CAP
    s0 = E1 * CAP + jnp.minimum(p0, CAP - 1)
    s1 = E2 * CAP + jnp.minimum(p1, CAP - 1)
    ds0_ref[...] = jnp.where(v0, s0, TRASH)
    ds1_ref[...] = jnp.where(v1, s1, TRASH)
    cs0_ref[...] = s0
    cs1_ref[...] = s1
    gg0_ref[...] = jnp.where(v0, G0, 0.0)
    gg1_ref[...] = jnp.where(v1, G1, 0.0)


_router = pl.pallas_call(
    _router_body,
    out_shape=(
        jax.ShapeDtypeStruct((16, 128), jnp.int32),
        jax.ShapeDtypeStruct((16, 128), jnp.int32),
        jax.ShapeDtypeStruct((16, 128), jnp.int32),
        jax.ShapeDtypeStruct((16, 128), jnp.int32),
        jax.ShapeDtypeStruct((16, 128), jnp.float32),
        jax.ShapeDtypeStruct((16, 128), jnp.float32),
    ),
)


_NC = 2
_NW = 32
_TPW = T // _NW


@functools.lru_cache(maxsize=None)
def _sc_dispatch():
    mesh = plsc.VectorSubcoreMesh(core_axis_name="c", subcore_axis_name="s")

    @functools.partial(
        pl.kernel,
        out_type=jax.ShapeDtypeStruct((NROW, D_MODEL), jnp.float32),
        mesh=mesh,
        scratch_types=[
            pltpu.VMEM((2, _TPW // 2), jnp.int32),
            pltpu.VMEM((2, _TPW // 2), jnp.int32),
            pltpu.VMEM((2, _TPW // 2, D_MODEL), jnp.float32),
            [pltpu.SemaphoreType.DMA, pltpu.SemaphoreType.DMA],
            pltpu.SemaphoreType.DMA,
        ],
    )
    def _dispatch(x_hbm, ds0_hbm, ds1_hbm, out_hbm, idx0_v, idx1_v, rows_v,
                  rsems, wsem):
        wid = jax.lax.axis_index("s") * _NC + jax.lax.axis_index("c")
        base = wid * _TPW
        half = _TPW // 2
        r0 = pltpu.async_copy(x_hbm.at[pl.ds(base, half)], rows_v.at[0],
                              rsems[0])
        r1 = pltpu.async_copy(x_hbm.at[pl.ds(base + half, half)], rows_v.at[1],
                              rsems[1])
        for h in range(2):
            pltpu.sync_copy(ds0_hbm.at[pl.ds(base + h * half, half)],
                            idx0_v.at[h])
            pltpu.sync_copy(ds1_hbm.at[pl.ds(base + h * half, half)],
                            idx1_v.at[h])
        copies = []
        for h in range(2):
            (r0 if h == 0 else r1).wait()
            copies.append(pltpu.async_copy(
                rows_v.at[h], out_hbm.at[idx0_v.at[h]], wsem))
            copies.append(pltpu.async_copy(
                rows_v.at[h], out_hbm.at[idx1_v.at[h]], wsem))
        for c in copies:
            c.wait()

    return _dispatch


_BF = 1024
_NF = D_FF // _BF


def _glu_body(x_ref, wa_ref, wb_ref, wo_ref, o_ref):
    f = pl.program_id(1)
    xb = x_ref[...].astype(jnp.bfloat16)
    wa = wa_ref[0, 0].astype(jnp.bfloat16)
    wb = wb_ref[0, 0].astype(jnp.bfloat16)
    hA = jax.lax.dot_general(xb, wa, (((1,), (1,)), ((), ())),
                             preferred_element_type=jnp.float32)
    hB = jax.lax.dot_general(xb, wb, (((1,), (1,)), ((), ())),
                             preferred_element_type=jnp.float32)
    act = 0.5 * hA * (1.0 + jax.lax.erf(hA * 0.7071067811865476)) * hB
    wo = wo_ref[0].astype(jnp.bfloat16)
    p = jax.lax.dot_general(act.astype(jnp.bfloat16), wo,
                            (((1,), (1,)), ((), ())),
                            preferred_element_type=jnp.float32)

    @pl.when(f == 0)
    def _():
        o_ref[...] = p

    @pl.when(f != 0)
    def _():
        o_ref[...] += p


_glu = pl.pallas_call(
    _glu_body,
    grid=(N_EXP, _NF),
    in_specs=[
        pl.BlockSpec((CAP, D_MODEL), lambda e, f: (e, 0)),
        pl.BlockSpec((1, 1, _BF, D_MODEL), lambda e, f: (e, 0, f, 0)),
        pl.BlockSpec((1, 1, _BF, D_MODEL), lambda e, f: (e, 1, f, 0)),
        pl.BlockSpec((1, D_MODEL, _BF), lambda e, f: (e, 0, f)),
    ],
    out_specs=pl.BlockSpec((CAP, D_MODEL), lambda e, f: (e, 0)),
    out_shape=jax.ShapeDtypeStruct((N_EXP * CAP, D_MODEL), jnp.float32),
    compiler_params=pltpu.CompilerParams(
        dimension_semantics=("arbitrary", "arbitrary"),
        vmem_limit_bytes=120 * 1024 * 1024),
)


_CH = 16


@functools.lru_cache(maxsize=None)
def _sc_combine():
    mesh = plsc.VectorSubcoreMesh(core_axis_name="c", subcore_axis_name="s")

    @functools.partial(
        pl.kernel,
        out_type=jax.ShapeDtypeStruct((T, D_MODEL), jnp.float32),
        mesh=mesh,
        scratch_types=[
            pltpu.VMEM((_TPW,), jnp.int32),
            pltpu.VMEM((_TPW,), jnp.int32),
            pltpu.VMEM((_TPW, 32), jnp.float32),
            pltpu.VMEM((2, _CH, D_MODEL), jnp.float32),
            pltpu.VMEM((2, _CH, D_MODEL), jnp.float32),
            [pltpu.SemaphoreType.DMA, pltpu.SemaphoreType.DMA],
        ],
    )
    def _combine(eo_hbm, cs0_hbm, cs1_hbm, g_hbm, out_hbm,
                 i0_v, i1_v, g_v, b0_v, b1_v, sems):
        wid = jax.lax.axis_index("s") * _NC + jax.lax.axis_index("c")
        base = wid * _TPW
        pltpu.sync_copy(cs0_hbm.at[pl.ds(base, _TPW)], i0_v)
        pltpu.sync_copy(cs1_hbm.at[pl.ds(base, _TPW)], i1_v)
        pltpu.sync_copy(g_hbm.at[pl.ds(base, _TPW)], g_v)
        nch = _TPW // _CH
        copies = []
        for h in range(nch):
            buf = h % 2
            c0 = pltpu.async_copy(
                eo_hbm.at[i0_v.at[pl.ds(h * _CH, _CH)]], b0_v.at[buf], sems[buf])
            c1 = pltpu.async_copy(
                eo_hbm.at[i1_v.at[pl.ds(h * _CH, _CH)]], b1_v.at[buf], sems[buf])
            copies.append((c0, c1))
            if h == 0:
                continue
            prev = h - 1
            pbuf = prev % 2
            copies[prev][0].wait()
            copies[prev][1].wait()

            def row_body(r, carry, pbuf=pbuf, prev=prev):
                ga = g_v[prev * _CH + r, pl.ds(0, 16)]
                gb = g_v[prev * _CH + r, pl.ds(16, 16)]
                for v in range(D_MODEL // 16):
                    sl = pl.ds(v * 16, 16)
                    b0_v[pbuf, r, sl] = (ga * b0_v[pbuf, r, sl]
                                         + gb * b1_v[pbuf, r, sl])
                return carry

            jax.lax.fori_loop(0, _CH, row_body, 0)
            pltpu.sync_copy(b0_v.at[pbuf],
                            out_hbm.at[pl.ds(base + prev * _CH, _CH)])
        last = nch - 1
        lbuf = last % 2
        copies[last][0].wait()
        copies[last][1].wait()

        def row_body_l(r, carry):
            ga = g_v[last * _CH + r, pl.ds(0, 16)]
            gb = g_v[last * _CH + r, pl.ds(16, 16)]
            for v in range(D_MODEL // 16):
                sl = pl.ds(v * 16, 16)
                b0_v[lbuf, r, sl] = (ga * b0_v[lbuf, r, sl]
                                     + gb * b1_v[lbuf, r, sl])
            return carry

        jax.lax.fori_loop(0, _CH, row_body_l, 0)
        pltpu.sync_copy(b0_v.at[lbuf],
                        out_hbm.at[pl.ds(base + last * _CH, _CH)])

    return _combine


def kernel(hidden_states, gate_w, w_in, w_out):
    x2d = hidden_states.reshape(T, D_MODEL)
    gwt = jnp.zeros((D_MODEL, 128), jnp.float32).at[:, :N_EXP].set(gate_w.T)
    ds0, ds1, cs0, cs1, gg0, gg1 = _router(x2d, gwt)
    expx = _sc_dispatch()(x2d, ds0.reshape(T), ds1.reshape(T))
    w_in4 = w_in.reshape(N_EXP, 2, D_FF, D_MODEL)
    eo = _glu(expx, w_in4, w_in4, w_out)
    gx = jnp.concatenate(
        [jnp.broadcast_to(gg0.reshape(T, 1), (T, 16)),
         jnp.broadcast_to(gg1.reshape(T, 1), (T, 16))], axis=1)
    out = _sc_combine()(eo, cs0.reshape(T), cs1.reshape(T), gx)
    return out.reshape(1, T, D_MODEL)


# speedup vs baseline: 1.0691x; 1.0012x over previous
import functools

import jax
import jax.numpy as jnp
from jax.experimental import pallas as pl
from jax.experimental.pallas import tpu as pltpu
from jax.experimental.pallas import tpu_sc as plsc

N_EXP = 8
TOP_K = 2
D_MODEL = 1024
D_FF = 2048
T = 2048
CAP = 640
NROW = 9 * CAP
TRASH = NROW - 1
NEG = -1e30


def _router_body(x_ref, gw_ref,
                 ds0_ref, ds1_ref, cs0_ref, cs1_ref,
                 gg0_ref, gg1_ref):
    logits = jax.lax.dot_general(
        x_ref[...].astype(jnp.bfloat16), gw_ref[...].astype(jnp.bfloat16),
        (((1,), (0,)), ((), ())),
        preferred_element_type=jnp.float32)
    lane = jax.lax.broadcasted_iota(jnp.int32, (T, 128), 1)
    lg = jnp.where(lane < N_EXP, logits, NEG)
    v1 = jnp.max(lg, axis=1, keepdims=True)
    e1 = jnp.min(jnp.where(lg == v1, lane, 127), axis=1, keepdims=True)
    lg2 = jnp.where(lane == e1, NEG, lg)
    v2 = jnp.max(lg2, axis=1, keepdims=True)
    e2 = jnp.min(jnp.where(lg2 == v2, lane, 127), axis=1, keepdims=True)
    g0 = jax.nn.sigmoid(v1 - v2)
    E1 = e1.reshape(16, 128)
    E2 = e2.reshape(16, 128)
    G0 = g0.reshape(16, 128)
    G1 = (1.0 - g0).reshape(16, 128)
    r1 = jax.lax.broadcasted_iota(jnp.int32, (128, 128), 0)
    c1 = jax.lax.broadcasted_iota(jnp.int32, (128, 128), 1)
    U = (r1 < c1).astype(jnp.float32)
    r2 = jax.lax.broadcasted_iota(jnp.int32, (16, 16), 0)
    c2 = jax.lax.broadcasted_iota(jnp.int32, (16, 16), 1)
    TL = (c2 < r2).astype(jnp.float32)

    Ws = []
    Scols = []
    for e in range(N_EXP):
        A = (E1 == e).astype(jnp.float32) + (E2 == e).astype(jnp.float32)
        Ws.append(jax.lax.dot_general(
            A, U, (((1,), (0,)), ((), ())),
            preferred_element_type=jnp.float32))
        Scols.append(jnp.sum(A, axis=1, keepdims=True))
    S = jnp.concatenate(Scols, axis=1)
    CP = jax.lax.dot_general(
        TL, S, (((1,), (0,)), ((), ())),
        preferred_element_type=jnp.float32)

    pos0 = jnp.zeros((16, 128), jnp.float32)
    pos1 = jnp.zeros((16, 128), jnp.float32)
    for e in range(N_EXP):
        cnt = Ws[e] + CP[:, e:e + 1]
        pos0 = pos0 + jnp.where(E1 == e, cnt, 0.0)
        pos1 = pos1 + jnp.where(E2 == e, cnt, 0.0)
    p0 = pos0.astype(jnp.int32)
    p1 = pos1.astype(jnp.int32)
    v0 = p0 < CAP
    v1 = p1 < CAP
    s0 = E1 * CAP + jnp.minimum(p0, CAP - 1)
    s1 = E2 * CAP + jnp.minimum(p1, CAP - 1)
    ds0_ref[...] = jnp.where(v0, s0, TRASH)
    ds1_ref[...] = jnp.where(v1, s1, TRASH)
    cs0_ref[...] = s0
    cs1_ref[...] = s1
    gg0_ref[...] = jnp.where(v0, G0, 0.0)
    gg1_ref[...] = jnp.where(v1, G1, 0.0)


_router = pl.pallas_call(
    _router_body,
    out_shape=(
        jax.ShapeDtypeStruct((16, 128), jnp.int32),
        jax.ShapeDtypeStruct((16, 128), jnp.int32),
        jax.ShapeDtypeStruct((16, 128), jnp.int32),
        jax.ShapeDtypeStruct((16, 128), jnp.int32),
        jax.ShapeDtypeStruct((16, 128), jnp.float32),
        jax.ShapeDtypeStruct((16, 128), jnp.float32),
    ),
)


_NC = 2
_NW = 32
_TPW = T // _NW


@functools.lru_cache(maxsize=None)
def _sc_dispatch():
    mesh = plsc.VectorSubcoreMesh(core_axis_name="c", subcore_axis_name="s")

    @functools.partial(
        pl.kernel,
        out_type=jax.ShapeDtypeStruct((NROW, D_MODEL), jnp.float32),
        mesh=mesh,
        scratch_types=[
            pltpu.VMEM((2, _TPW // 2), jnp.int32),
            pltpu.VMEM((2, _TPW // 2), jnp.int32),
            pltpu.VMEM((2, _TPW // 2, D_MODEL), jnp.float32),
            [pltpu.SemaphoreType.DMA, pltpu.SemaphoreType.DMA],
            pltpu.SemaphoreType.DMA,
        ],
    )
    def _dispatch(x_hbm, ds0_hbm, ds1_hbm, out_hbm, idx0_v, idx1_v, rows_v,
                  rsems, wsem):
        wid = jax.lax.axis_index("s") * _NC + jax.lax.axis_index("c")
        base = wid * _TPW
        half = _TPW // 2
        r0 = pltpu.async_copy(x_hbm.at[pl.ds(base, half)], rows_v.at[0],
                              rsems[0])
        r1 = pltpu.async_copy(x_hbm.at[pl.ds(base + half, half)], rows_v.at[1],
                              rsems[1])
        for h in range(2):
            pltpu.sync_copy(ds0_hbm.at[pl.ds(base + h * half, half)],
                            idx0_v.at[h])
            pltpu.sync_copy(ds1_hbm.at[pl.ds(base + h * half, half)],
                            idx1_v.at[h])
        copies = []
        for h in range(2):
            (r0 if h == 0 else r1).wait()
            copies.append(pltpu.async_copy(
                rows_v.at[h], out_hbm.at[idx0_v.at[h]], wsem))
            copies.append(pltpu.async_copy(
                rows_v.at[h], out_hbm.at[idx1_v.at[h]], wsem))
        for c in copies:
            c.wait()

    return _dispatch


_BF = 1024
_NF = D_FF // _BF


def _glu_body(x_ref, wa_ref, wb_ref, wo_ref, o_ref):
    f = pl.program_id(1)
    xb = x_ref[...].astype(jnp.bfloat16)
    wa = wa_ref[0, 0].astype(jnp.bfloat16)
    wb = wb_ref[0, 0].astype(jnp.bfloat16)
    hA = jax.lax.dot_general(xb, wa, (((1,), (1,)), ((), ())),
                             preferred_element_type=jnp.float32)
    hB = jax.lax.dot_general(xb, wb, (((1,), (1,)), ((), ())),
                             preferred_element_type=jnp.float32)
    act = 0.5 * hA * (1.0 + jax.lax.erf(hA * 0.7071067811865476)) * hB
    wo = wo_ref[0].astype(jnp.bfloat16)
    p = jax.lax.dot_general(act.astype(jnp.bfloat16), wo,
                            (((1,), (1,)), ((), ())),
                            preferred_element_type=jnp.float32)

    @pl.when(f == 0)
    def _():
        o_ref[...] = p

    @pl.when(f != 0)
    def _():
        o_ref[...] += p


_glu = pl.pallas_call(
    _glu_body,
    grid=(N_EXP, _NF),
    in_specs=[
        pl.BlockSpec((CAP, D_MODEL), lambda e, f: (e, 0)),
        pl.BlockSpec((1, 1, _BF, D_MODEL), lambda e, f: (e, 0, f, 0)),
        pl.BlockSpec((1, 1, _BF, D_MODEL), lambda e, f: (e, 1, f, 0)),
        pl.BlockSpec((1, D_MODEL, _BF), lambda e, f: (e, 0, f)),
    ],
    out_specs=pl.BlockSpec((CAP, D_MODEL), lambda e, f: (e, 0)),
    out_shape=jax.ShapeDtypeStruct((N_EXP * CAP, D_MODEL), jnp.float32),
    compiler_params=pltpu.CompilerParams(
        dimension_semantics=("arbitrary", "arbitrary"),
        vmem_limit_bytes=120 * 1024 * 1024),
)


_CH = 16


@functools.lru_cache(maxsize=None)
def _sc_combine():
    mesh = plsc.VectorSubcoreMesh(core_axis_name="c", subcore_axis_name="s")

    @functools.partial(
        pl.kernel,
        out_type=jax.ShapeDtypeStruct((T, D_MODEL), jnp.float32),
        mesh=mesh,
        scratch_types=[
            pltpu.VMEM((_TPW,), jnp.int32),
            pltpu.VMEM((_TPW,), jnp.int32),
            pltpu.VMEM((_TPW, 32), jnp.float32),
            pltpu.VMEM((3, _CH, D_MODEL), jnp.float32),
            pltpu.VMEM((3, _CH, D_MODEL), jnp.float32),
            [pltpu.SemaphoreType.DMA, pltpu.SemaphoreType.DMA,
             pltpu.SemaphoreType.DMA],
        ],
    )
    def _combine(eo_hbm, cs0_hbm, cs1_hbm, g_hbm, out_hbm,
                 i0_v, i1_v, g_v, b0_v, b1_v, sems):
        wid = jax.lax.axis_index("s") * _NC + jax.lax.axis_index("c")
        base = wid * _TPW
        pltpu.sync_copy(cs0_hbm.at[pl.ds(base, _TPW)], i0_v)
        pltpu.sync_copy(cs1_hbm.at[pl.ds(base, _TPW)], i1_v)
        pltpu.sync_copy(g_hbm.at[pl.ds(base, _TPW)], g_v)
        nch = _TPW // _CH
        copies = []
        for h in range(nch):
            buf = h % 3
            c0 = pltpu.async_copy(
                eo_hbm.at[i0_v.at[pl.ds(h * _CH, _CH)]], b0_v.at[buf], sems[buf])
            c1 = pltpu.async_copy(
                eo_hbm.at[i1_v.at[pl.ds(h * _CH, _CH)]], b1_v.at[buf], sems[buf])
            copies.append((c0, c1))
            if h == 0:
                continue
            prev = h - 1
            pbuf = prev % 3
            copies[prev][0].wait()
            copies[prev][1].wait()

            def row_body(r, carry, pbuf=pbuf, prev=prev):
                ga = g_v[prev * _CH + r, pl.ds(0, 16)]
                gb = g_v[prev * _CH + r, pl.ds(16, 16)]
                for v in range(D_MODEL // 16):
                    sl = pl.ds(v * 16, 16)
                    b0_v[pbuf, r, sl] = (ga * b0_v[pbuf, r, sl]
                                         + gb * b1_v[pbuf, r, sl])
                return carry

            jax.lax.fori_loop(0, _CH, row_body, 0)
            pltpu.sync_copy(b0_v.at[pbuf],
                            out_hbm.at[pl.ds(base + prev * _CH, _CH)])
        last = nch - 1
        lbuf = last % 3
        copies[last][0].wait()
        copies[last][1].wait()

        def row_body_l(r, carry):
            ga = g_v[last * _CH + r, pl.ds(0, 16)]
            gb = g_v[last * _CH + r, pl.ds(16, 16)]
            for v in range(D_MODEL // 16):
                sl = pl.ds(v * 16, 16)
                b0_v[lbuf, r, sl] = (ga * b0_v[lbuf, r, sl]
                                     + gb * b1_v[lbuf, r, sl])
            return carry

        jax.lax.fori_loop(0, _CH, row_body_l, 0)
        pltpu.sync_copy(b0_v.at[lbuf],
                        out_hbm.at[pl.ds(base + last * _CH, _CH)])

    return _combine


def kernel(hidden_states, gate_w, w_in, w_out):
    x2d = hidden_states.reshape(T, D_MODEL)
    gwt = jnp.zeros((D_MODEL, 128), jnp.float32).at[:, :N_EXP].set(gate_w.T)
    ds0, ds1, cs0, cs1, gg0, gg1 = _router(x2d, gwt)
    expx = _sc_dispatch()(x2d, ds0.reshape(T), ds1.reshape(T))
    w_in4 = w_in.reshape(N_EXP, 2, D_FF, D_MODEL)
    eo = _glu(expx, w_in4, w_in4, w_out)
    gx = jnp.concatenate(
        [jnp.broadcast_to(gg0.reshape(T, 1), (T, 16)),
         jnp.broadcast_to(gg1.reshape(T, 1), (T, 16))], axis=1)
    out = _sc_combine()(eo, cs0.reshape(T), cs1.reshape(T), gx)
    return out.reshape(1, T, D_MODEL)
